# Initial kernel scaffold; baseline (speedup 1.0000x reference)
import jax
import jax.numpy as jnp
from jax.experimental import pallas as pl


def kernel(x, edge_index, W1l, b1l, W1r, W2l, b2l, W2r, Wlin, blin):
    raise NotImplementedError("write your pallas kernel here")


# speedup vs baseline: 7.1102x; 7.1102x over previous
import functools

import jax
import jax.numpy as jnp
from jax import lax
from jax.experimental import pallas as pl
from jax.experimental.pallas import tpu as pltpu
from jax.experimental.pallas import tpu_sc as plsc

N = 10000
E = 320000
D = 128
H = 128
OUT = 64

NC = 2
NS = 16
NW = NC * NS
EPT = E // NW
K = 80
NCHUNK = EPT // K
NPAD = 10240
ROWB = 640
ZCH = 80

BN = 1000
GRID = N // BN


def _sc_mesh():
    return plsc.VectorSubcoreMesh(
        core_axis_name="c", subcore_axis_name="s",
        num_cores=NC, num_subcores=NS)


def _sc_agg_cnt_body(feat, srcI, dstI, z2d, z1d, agg_out, cnt_out,
                     rows, ones, sidxV, didxV, accS, cntS, sem):
    cid = lax.axis_index("c")
    sid = lax.axis_index("s")
    wid = cid * NS + sid

    pltpu.sync_copy(srcI.at[wid], sidxV)
    pltpu.sync_copy(dstI.at[wid], didxV)

    for j in range(K // 16):
        ones[pl.ds(j * 16, 16)] = jnp.ones((16,), jnp.float32)

    lo = sid * ROWB
    nzc = jnp.minimum((N - lo) // ZCH, ROWB // ZCH)

    def _zero(i, _):
        pltpu.sync_copy(z2d, accS.at[pl.ds(lo + i * ZCH, ZCH)])
        return 0
    lax.fori_loop(0, nzc, _zero, 0)
    pltpu.sync_copy(z1d, cntS.at[pl.ds(lo, ROWB)])
    plsc.subcore_barrier()

    def _edge(i, _):
        pltpu.async_copy(feat.at[sidxV.at[i]], rows, sem).wait()
        pltpu.sync_copy(rows, accS.at[didxV.at[i]], add=True)
        pltpu.sync_copy(ones, cntS.at[didxV.at[i]], add=True)
        return 0
    lax.fori_loop(0, NCHUNK, _edge, 0)
    plsc.subcore_barrier()

    def _wout(i, _):
        pltpu.sync_copy(accS.at[pl.ds(lo + i * ZCH, ZCH)],
                        agg_out.at[cid, pl.ds(lo + i * ZCH, ZCH)])
        return 0
    lax.fori_loop(0, nzc, _wout, 0)
    pltpu.sync_copy(cntS.at[pl.ds(lo, ROWB)], cnt_out.at[cid, pl.ds(lo, ROWB)])


def _sc_agg_body(feat, srcI, dstI, z2d, agg_out, rows, sidxV, didxV, accS, sem):
    cid = lax.axis_index("c")
    sid = lax.axis_index("s")
    wid = cid * NS + sid

    pltpu.sync_copy(srcI.at[wid], sidxV)
    pltpu.sync_copy(dstI.at[wid], didxV)

    lo = sid * ROWB
    nzc = jnp.minimum((N - lo) // ZCH, ROWB // ZCH)

    def _zero(i, _):
        pltpu.sync_copy(z2d, accS.at[pl.ds(lo + i * ZCH, ZCH)])
        return 0
    lax.fori_loop(0, nzc, _zero, 0)
    plsc.subcore_barrier()

    def _edge(i, _):
        pltpu.async_copy(feat.at[sidxV.at[i]], rows, sem).wait()
        pltpu.sync_copy(rows, accS.at[didxV.at[i]], add=True)
        return 0
    lax.fori_loop(0, NCHUNK, _edge, 0)
    plsc.subcore_barrier()

    def _wout(i, _):
        pltpu.sync_copy(accS.at[pl.ds(lo + i * ZCH, ZCH)],
                        agg_out.at[cid, pl.ds(lo + i * ZCH, ZCH)])
        return 0
    lax.fori_loop(0, nzc, _wout, 0)


_sc_agg_cnt = functools.partial(
    pl.kernel,
    out_type=[jax.ShapeDtypeStruct((NC, N, H), jnp.float32),
              jax.ShapeDtypeStruct((NC, NPAD), jnp.float32)],
    mesh=_sc_mesh(),
    scratch_types=[
        pltpu.VMEM((K, H), jnp.float32),
        pltpu.VMEM((K,), jnp.float32),
        pltpu.VMEM((NCHUNK, K), jnp.int32),
        pltpu.VMEM((NCHUNK, K), jnp.int32),
        pltpu.VMEM_SHARED((N, H), jnp.float32),
        pltpu.VMEM_SHARED((NPAD,), jnp.float32),
        pltpu.SemaphoreType.DMA,
    ],
)(_sc_agg_cnt_body)

_sc_agg = functools.partial(
    pl.kernel,
    out_type=jax.ShapeDtypeStruct((NC, N, H), jnp.float32),
    mesh=_sc_mesh(),
    scratch_types=[
        pltpu.VMEM((K, H), jnp.float32),
        pltpu.VMEM((NCHUNK, K), jnp.int32),
        pltpu.VMEM((NCHUNK, K), jnp.int32),
        pltpu.VMEM_SHARED((N, H), jnp.float32),
        pltpu.SemaphoreType.DMA,
    ],
)(_sc_agg_body)


def _pre_body(x_ref, wl_ref, wr_ref, b_ref, xl_ref, xr_ref):
    xb = x_ref[...]
    xl_ref[...] = jnp.dot(xb, wl_ref[...], preferred_element_type=jnp.float32)
    xr_ref[...] = (jnp.dot(xb, wr_ref[...], preferred_element_type=jnp.float32)
                   + b_ref[...])


def _mid_body(agg_ref, cnt_ref, xr_ref, wl_ref, wr_ref, b_ref,
              yl_ref, yr_ref):
    s = agg_ref[0] + agg_ref[1]
    c = cnt_ref[0] + cnt_ref[1]
    mean = s / jnp.maximum(c, 1.0)
    h = jnp.maximum(mean + xr_ref[...], 0.0)
    yl_ref[...] = jnp.dot(h, wl_ref[...], preferred_element_type=jnp.float32)
    yr_ref[...] = (jnp.dot(h, wr_ref[...], preferred_element_type=jnp.float32)
                   + b_ref[...])


def _fin_body(agg_ref, cnt_ref, yr_ref, w_ref, b_ref, o_ref):
    s = agg_ref[0] + agg_ref[1]
    c = cnt_ref[0] + cnt_ref[1]
    mean = s / jnp.maximum(c, 1.0)
    h = jnp.maximum(mean + yr_ref[...], 0.0)
    o_ref[...] = (jnp.dot(h, w_ref[...], preferred_element_type=jnp.float32)
                  + b_ref[...])


def _tc_pre(x, wlT, wrT, b):
    return pl.pallas_call(
        _pre_body,
        grid=(GRID,),
        in_specs=[
            pl.BlockSpec((BN, D), lambda i: (i, 0)),
            pl.BlockSpec((D, H), lambda i: (0, 0)),
            pl.BlockSpec((D, H), lambda i: (0, 0)),
            pl.BlockSpec((1, H), lambda i: (0, 0)),
        ],
        out_specs=[
            pl.BlockSpec((BN, H), lambda i: (i, 0)),
            pl.BlockSpec((BN, H), lambda i: (i, 0)),
        ],
        out_shape=[
            jax.ShapeDtypeStruct((N, H), jnp.float32),
            jax.ShapeDtypeStruct((N, H), jnp.float32),
        ],
    )(x, wlT, wrT, b)


def _tc_mid(agg, cnt3, xr, wlT, wrT, b):
    return pl.pallas_call(
        _mid_body,
        grid=(GRID,),
        in_specs=[
            pl.BlockSpec((NC, BN, H), lambda i: (0, i, 0)),
            pl.BlockSpec((NC, BN, 1), lambda i: (0, i, 0)),
            pl.BlockSpec((BN, H), lambda i: (i, 0)),
            pl.BlockSpec((H, H), lambda i: (0, 0)),
            pl.BlockSpec((H, H), lambda i: (0, 0)),
            pl.BlockSpec((1, H), lambda i: (0, 0)),
        ],
        out_specs=[
            pl.BlockSpec((BN, H), lambda i: (i, 0)),
            pl.BlockSpec((BN, H), lambda i: (i, 0)),
        ],
        out_shape=[
            jax.ShapeDtypeStruct((N, H), jnp.float32),
            jax.ShapeDtypeStruct((N, H), jnp.float32),
        ],
    )(agg, cnt3, xr, wlT, wrT, b)


def _tc_fin(agg, cnt3, yr, wT, b):
    return pl.pallas_call(
        _fin_body,
        grid=(GRID,),
        in_specs=[
            pl.BlockSpec((NC, BN, H), lambda i: (0, i, 0)),
            pl.BlockSpec((NC, BN, 1), lambda i: (0, i, 0)),
            pl.BlockSpec((BN, H), lambda i: (i, 0)),
            pl.BlockSpec((H, OUT), lambda i: (0, 0)),
            pl.BlockSpec((1, OUT), lambda i: (0, 0)),
        ],
        out_specs=pl.BlockSpec((BN, OUT), lambda i: (i, 0)),
        out_shape=jax.ShapeDtypeStruct((N, OUT), jnp.float32),
    )(agg, cnt3, yr, wT, b)


def kernel(x, edge_index, W1l, b1l, W1r, W2l, b2l, W2r, Wlin, blin):
    src = edge_index[0].reshape(NW, NCHUNK, K)
    dst = edge_index[1].reshape(NW, NCHUNK, K)
    z2d = jnp.zeros((ZCH, H), jnp.float32)
    z1d = jnp.zeros((ROWB,), jnp.float32)

    xl, xr = _tc_pre(x, W1l.T, W1r.T, b1l.reshape(1, H))
    agg1, cnt = _sc_agg_cnt(xl, src, dst, z2d, z1d)
    cnt3 = cnt.reshape(NC, NPAD, 1)
    yl, yr = _tc_mid(agg1, cnt3, xr, W2l.T, W2r.T, b2l.reshape(1, H))
    agg2 = _sc_agg(yl, src, dst, z2d)
    return _tc_fin(agg2, cnt3, yr, Wlin.T, blin.reshape(1, OUT))


# speedup vs baseline: 12.4257x; 1.7476x over previous
import functools

import jax
import jax.numpy as jnp
from jax import lax
from jax.experimental import pallas as pl
from jax.experimental.pallas import tpu as pltpu
from jax.experimental.pallas import tpu_sc as plsc

N = 10000
E = 320000
D = 128
H = 128
OUT = 64

NC = 2
NS = 16
HC = H // NC
EPT = E // NS
K = 80
NCHUNK = EPT // K
NB = 5
NPAD = 10240
ROWB = 640
ZCH = 80

BN = 1000
GRID = N // BN


def _sc_mesh():
    return plsc.VectorSubcoreMesh(
        core_axis_name="c", subcore_axis_name="s",
        num_cores=NC, num_subcores=NS)


def _zero_acc(z2d, accS, lo, nzc, zsem):
    def _issue(i, _):
        pltpu.async_copy(z2d, accS.at[pl.ds(lo + i * ZCH, ZCH)], zsem)
        return 0
    lax.fori_loop(0, nzc, _issue, 0)

    def _drain(i, _):
        pltpu.make_async_copy(z2d, accS.at[pl.ds(lo + i * ZCH, ZCH)],
                              zsem).wait()
        return 0
    lax.fori_loop(0, nzc, _drain, 0)


def _writeout(accS, agg_out, cid, lo, nzc, wsem):
    def _issue(i, _):
        pltpu.async_copy(accS.at[pl.ds(lo + i * ZCH, ZCH)],
                         agg_out.at[cid, pl.ds(lo + i * ZCH, ZCH)], wsem)
        return 0
    lax.fori_loop(0, nzc, _issue, 0)

    def _drain(i, _):
        pltpu.make_async_copy(accS.at[pl.ds(lo + i * ZCH, ZCH)],
                              agg_out.at[cid, pl.ds(lo + i * ZCH, ZCH)],
                              wsem).wait()
        return 0
    lax.fori_loop(0, nzc, _drain, 0)


def _edge_pipeline(feat, sidxV, didxV, accS, rowbufs, gsems, ssems, cnt_pair):
    for b in range(NB - 1):
        pltpu.async_copy(feat.at[sidxV.at[b]], rowbufs[b], gsems[b])

    def _outer(i, _):
        for b in range(NB):
            c = i * NB + b
            nb_ = (b + NB - 1) % NB

            @pl.when(c + NB - 1 < NCHUNK)
            def _issue_ahead():
                @pl.when(c >= 1)
                def _wait_prev_scatter():
                    pltpu.make_async_copy(
                        rowbufs[nb_], accS.at[didxV.at[0]], ssems[nb_]).wait()
                pltpu.async_copy(
                    feat.at[sidxV.at[c + NB - 1]], rowbufs[nb_], gsems[nb_])

            pltpu.make_async_copy(
                feat.at[sidxV.at[c]], rowbufs[b], gsems[b]).wait()
            pltpu.async_copy(
                rowbufs[b], accS.at[didxV.at[c]], ssems[b], add=True)
            if cnt_pair is not None:
                ones, cntS, csem = cnt_pair
                pltpu.async_copy(ones, cntS.at[didxV.at[c]], csem, add=True)
        return 0
    lax.fori_loop(0, NCHUNK // NB, _outer, 0)

    for b in range(NB):
        pltpu.make_async_copy(
            rowbufs[b], accS.at[didxV.at[0]], ssems[b]).wait()


def _sc_agg_cnt_body(feat0, feat1, srcI, dstI, z2d, z1d, agg_out, cnt_out,
                     r0, r1, r2, r3, r4, ones, sidxV, didxV, accS, cntS,
                     g0, g1, g2, g3, g4, s0, s1, s2, s3, s4, csem):
    rowbufs = [r0, r1, r2, r3, r4]
    gsems = [g0, g1, g2, g3, g4]
    ssems = [s0, s1, s2, s3, s4]
    cid = lax.axis_index("c")
    sid = lax.axis_index("s")

    pltpu.async_copy(srcI.at[sid], sidxV, g0)
    pltpu.async_copy(dstI.at[sid], didxV, g1)

    for j in range(K // 16):
        ones[pl.ds(j * 16, 16)] = jnp.ones((16,), jnp.float32)

    lo = sid * ROWB
    nzc = jnp.minimum((N - lo) // ZCH, ROWB // ZCH)
    _zero_acc(z2d, accS, lo, nzc, s0)

    @pl.when(cid == 0)
    def _zero_cnt():
        pltpu.sync_copy(z1d, cntS.at[pl.ds(lo, ROWB)])

    pltpu.make_async_copy(srcI.at[sid], sidxV, g0).wait()
    pltpu.make_async_copy(dstI.at[sid], didxV, g1).wait()
    plsc.subcore_barrier()

    @pl.when(cid == 0)
    def _run0():
        _edge_pipeline(feat0, sidxV, didxV, accS, rowbufs, gsems, ssems,
                       (ones, cntS, csem))
        pltpu.make_async_copy(srcI.at[sid], sidxV, csem).wait()

    @pl.when(cid == 1)
    def _run1():
        _edge_pipeline(feat1, sidxV, didxV, accS, rowbufs, gsems, ssems,
                       None)

    plsc.subcore_barrier()

    _writeout(accS, agg_out, cid, lo, nzc, g0)

    @pl.when(cid == 0)
    def _cnt_out():
        pltpu.sync_copy(cntS.at[pl.ds(lo, ROWB)], cnt_out.at[pl.ds(lo, ROWB)])


def _sc_agg_body(feat0, feat1, srcI, dstI, z2d, agg_out,
                 r0, r1, r2, r3, r4, sidxV, didxV, accS,
                 g0, g1, g2, g3, g4, s0, s1, s2, s3, s4):
    rowbufs = [r0, r1, r2, r3, r4]
    gsems = [g0, g1, g2, g3, g4]
    ssems = [s0, s1, s2, s3, s4]
    cid = lax.axis_index("c")
    sid = lax.axis_index("s")

    pltpu.async_copy(srcI.at[sid], sidxV, g0)
    pltpu.async_copy(dstI.at[sid], didxV, g1)

    lo = sid * ROWB
    nzc = jnp.minimum((N - lo) // ZCH, ROWB // ZCH)
    _zero_acc(z2d, accS, lo, nzc, s0)
    pltpu.make_async_copy(srcI.at[sid], sidxV, g0).wait()
    pltpu.make_async_copy(dstI.at[sid], didxV, g1).wait()
    plsc.subcore_barrier()

    @pl.when(cid == 0)
    def _run0():
        _edge_pipeline(feat0, sidxV, didxV, accS, rowbufs, gsems, ssems,
                       None)

    @pl.when(cid == 1)
    def _run1():
        _edge_pipeline(feat1, sidxV, didxV, accS, rowbufs, gsems, ssems,
                       None)

    plsc.subcore_barrier()
    _writeout(accS, agg_out, cid, lo, nzc, g0)


_sc_agg_cnt = functools.partial(
    pl.kernel,
    out_type=[jax.ShapeDtypeStruct((NC, N, HC), jnp.float32),
              jax.ShapeDtypeStruct((NPAD,), jnp.float32)],
    mesh=_sc_mesh(),
    compiler_params=pltpu.CompilerParams(use_tc_tiling_on_sc=False),
    scratch_types=(
        [pltpu.VMEM((K, HC), jnp.float32)] * NB
        + [pltpu.VMEM((K,), jnp.float32),
           pltpu.VMEM((NCHUNK, K), jnp.int32),
           pltpu.VMEM((NCHUNK, K), jnp.int32),
           pltpu.VMEM_SHARED((N, HC), jnp.float32),
           pltpu.VMEM_SHARED((NPAD,), jnp.float32)]
        + [pltpu.SemaphoreType.DMA] * (2 * NB + 1)
    ),
)(_sc_agg_cnt_body)

_sc_agg = functools.partial(
    pl.kernel,
    out_type=jax.ShapeDtypeStruct((NC, N, HC), jnp.float32),
    mesh=_sc_mesh(),
    compiler_params=pltpu.CompilerParams(use_tc_tiling_on_sc=False),
    scratch_types=(
        [pltpu.VMEM((K, HC), jnp.float32)] * NB
        + [pltpu.VMEM((NCHUNK, K), jnp.int32),
           pltpu.VMEM((NCHUNK, K), jnp.int32),
           pltpu.VMEM_SHARED((N, HC), jnp.float32)]
        + [pltpu.SemaphoreType.DMA] * (2 * NB)
    ),
)(_sc_agg_body)


def _pre_body(x_ref, wl_ref, wr_ref, b_ref, xl0_ref, xl1_ref, xr_ref):
    xb = x_ref[...]
    xl = jnp.dot(xb, wl_ref[...], preferred_element_type=jnp.float32)
    xl0_ref[...] = xl[:, :HC]
    xl1_ref[...] = xl[:, HC:]
    xr_ref[...] = (jnp.dot(xb, wr_ref[...], preferred_element_type=jnp.float32)
                   + b_ref[...])


def _mid_body(agg_ref, cnt_ref, xr_ref, wl_ref, wr_ref, b_ref,
              yl0_ref, yl1_ref, yr_ref):
    s = jnp.concatenate([agg_ref[0], agg_ref[1]], axis=1)
    mean = s / jnp.maximum(cnt_ref[...], 1.0)
    h = jnp.maximum(mean + xr_ref[...], 0.0)
    yl = jnp.dot(h, wl_ref[...], preferred_element_type=jnp.float32)
    yl0_ref[...] = yl[:, :HC]
    yl1_ref[...] = yl[:, HC:]
    yr_ref[...] = (jnp.dot(h, wr_ref[...], preferred_element_type=jnp.float32)
                   + b_ref[...])


def _fin_body(agg_ref, cnt_ref, yr_ref, w_ref, b_ref, o_ref):
    s = jnp.concatenate([agg_ref[0], agg_ref[1]], axis=1)
    mean = s / jnp.maximum(cnt_ref[...], 1.0)
    h = jnp.maximum(mean + yr_ref[...], 0.0)
    o_ref[...] = (jnp.dot(h, w_ref[...], preferred_element_type=jnp.float32)
                  + b_ref[...])


def _tc_pre(x, wlT, wrT, b):
    return pl.pallas_call(
        _pre_body,
        grid=(GRID,),
        in_specs=[
            pl.BlockSpec((BN, D), lambda i: (i, 0)),
            pl.BlockSpec((D, H), lambda i: (0, 0)),
            pl.BlockSpec((D, H), lambda i: (0, 0)),
            pl.BlockSpec((1, H), lambda i: (0, 0)),
        ],
        out_specs=[
            pl.BlockSpec((BN, HC), lambda i: (i, 0)),
            pl.BlockSpec((BN, HC), lambda i: (i, 0)),
            pl.BlockSpec((BN, H), lambda i: (i, 0)),
        ],
        out_shape=[
            jax.ShapeDtypeStruct((N, HC), jnp.float32),
            jax.ShapeDtypeStruct((N, HC), jnp.float32),
            jax.ShapeDtypeStruct((N, H), jnp.float32),
        ],
    )(x, wlT, wrT, b)


def _tc_mid(agg, cnt2, xr, wlT, wrT, b):
    return pl.pallas_call(
        _mid_body,
        grid=(GRID,),
        in_specs=[
            pl.BlockSpec((NC, BN, HC), lambda i: (0, i, 0)),
            pl.BlockSpec((BN, 1), lambda i: (i, 0)),
            pl.BlockSpec((BN, H), lambda i: (i, 0)),
            pl.BlockSpec((H, H), lambda i: (0, 0)),
            pl.BlockSpec((H, H), lambda i: (0, 0)),
            pl.BlockSpec((1, H), lambda i: (0, 0)),
        ],
        out_specs=[
            pl.BlockSpec((BN, HC), lambda i: (i, 0)),
            pl.BlockSpec((BN, HC), lambda i: (i, 0)),
            pl.BlockSpec((BN, H), lambda i: (i, 0)),
        ],
        out_shape=[
            jax.ShapeDtypeStruct((N, HC), jnp.float32),
            jax.ShapeDtypeStruct((N, HC), jnp.float32),
            jax.ShapeDtypeStruct((N, H), jnp.float32),
        ],
    )(agg, cnt2, xr, wlT, wrT, b)


def _tc_fin(agg, cnt2, yr, wT, b):
    return pl.pallas_call(
        _fin_body,
        grid=(GRID,),
        in_specs=[
            pl.BlockSpec((NC, BN, HC), lambda i: (0, i, 0)),
            pl.BlockSpec((BN, 1), lambda i: (i, 0)),
            pl.BlockSpec((BN, H), lambda i: (i, 0)),
            pl.BlockSpec((H, OUT), lambda i: (0, 0)),
            pl.BlockSpec((1, OUT), lambda i: (0, 0)),
        ],
        out_specs=pl.BlockSpec((BN, OUT), lambda i: (i, 0)),
        out_shape=jax.ShapeDtypeStruct((N, OUT), jnp.float32),
    )(agg, cnt2, yr, wT, b)


def kernel(x, edge_index, W1l, b1l, W1r, W2l, b2l, W2r, Wlin, blin):
    src = edge_index[0].reshape(NS, NCHUNK, K)
    dst = edge_index[1].reshape(NS, NCHUNK, K)
    z2d = jnp.zeros((ZCH, HC), jnp.float32)
    z1d = jnp.zeros((ROWB,), jnp.float32)

    xl0, xl1, xr = _tc_pre(x, W1l.T, W1r.T, b1l.reshape(1, H))
    agg1, cnt = _sc_agg_cnt(xl0, xl1, src, dst, z2d, z1d)
    cnt2 = cnt.reshape(NPAD, 1)
    yl0, yl1, yr = _tc_mid(agg1, cnt2, xr, W2l.T, W2r.T, b2l.reshape(1, H))
    agg2 = _sc_agg(yl0, yl1, src, dst, z2d)
    return _tc_fin(agg2, cnt2, yr, Wlin.T, blin.reshape(1, OUT))
